# Initial kernel scaffold; baseline (speedup 1.0000x reference)
import jax
import jax.numpy as jnp
from jax.experimental import pallas as pl


def kernel(x):
    raise NotImplementedError("write your pallas kernel here")


# speedup vs baseline: 1.3622x; 1.3622x over previous
import jax
import jax.numpy as jnp
from jax import lax
from jax.experimental import pallas as pl
from jax.experimental.pallas import tpu as pltpu

B = 64
C = 2
F = 599
T = 512
FC_IDX = 219
MID_LO = 199
MID_HI = 239
NOISE_COUNT = (F - (MID_HI - MID_LO + 1)) * T

RB1 = 32
G1 = (F + RB1 - 1) // RB1
RB3 = 16
G3 = (F + RB3 - 1) // RB3


def _stats_kernel(x_ref, rowfc_ref, sums_ref):
    i = pl.program_id(0)
    rows = lax.broadcasted_iota(jnp.int32, (RB1, T), 0) + i * RB1
    mag = jnp.abs(x_ref[:, 0, :, :]) + jnp.abs(x_ref[:, 1, :, :])
    valid = (rows < F)[None, :, :]
    midm = ((rows >= MID_LO) & (rows <= MID_HI))[None, :, :]
    fcm = (rows == FC_IDX)[None, :, :]
    zero = jnp.zeros_like(mag)
    s_all = jnp.sum(jnp.where(valid, mag, zero), axis=(1, 2))
    s_mid = jnp.sum(jnp.where(midm, mag, zero), axis=(1, 2))
    fc_part = jnp.sum(jnp.where(fcm, mag, zero), axis=1)

    @pl.when(i == 0)
    def _():
        rowfc_ref[...] = fc_part
        sums_ref[:, 0:1] = s_all[:, None]
        sums_ref[:, 1:2] = s_mid[:, None]
        sums_ref[:, 2:8] = jnp.zeros((B, 6), jnp.float32)

    @pl.when(i > 0)
    def _():
        rowfc_ref[...] = rowfc_ref[...] + fc_part
        sums_ref[:, 0:1] = sums_ref[:, 0:1] + s_all[:, None]
        sums_ref[:, 1:2] = sums_ref[:, 1:2] + s_mid[:, None]


def _mask_kernel(rowfc_ref, sums_ref, h_ref, band_ref):
    row_fc = rowfc_ref[...]
    tcol = lax.broadcasted_iota(jnp.int32, (B, T), 1)
    m = jnp.max(row_fc, axis=1, keepdims=True)
    mid = jnp.min(jnp.where(row_fc == m, tcol, T), axis=1, keepdims=True)
    w20 = (tcol >= jnp.maximum(mid - 20, 0)) & (tcol < mid + 20)
    sig = jnp.sum(jnp.where(w20, row_fc, jnp.zeros_like(row_fc)), axis=1,
                  keepdims=True)
    noise = (sums_ref[:, 0:1] - sums_ref[:, 1:2]) * (1.0 / NOISE_COUNT)
    snr = 10.0 * (jnp.log((sig - noise) ** 2 / noise ** 2) /
                  jnp.log(jnp.float32(10.0)))
    hb = jnp.maximum(jnp.trunc(6.0 * (snr - 48.0) + 27.0).astype(jnp.int32),
                     8)
    cover = (tcol >= mid - 8) & (tcol < jnp.minimum(mid + 8, T))
    h = jnp.max(jnp.where(cover, hb, 0), axis=0, keepdims=True)
    h_ref[...] = h
    hbmax = jnp.max(hb)
    band_lo = jnp.maximum(FC_IDX - hbmax, 0)
    band_hi = jnp.minimum(FC_IDX + hbmax, F)
    band_ref[0] = band_lo // RB3
    band_ref[1] = (band_hi - 1) // RB3


def _apply_kernel(band_ref, x_ref, h_ref, out_ref):
    i = pl.program_id(0)
    rows = lax.broadcasted_iota(jnp.int32, (RB3, T), 0) + i * RB3
    h = h_ref[...]
    mask = (rows >= FC_IDX - h) & (rows < FC_IDX + h)
    x = x_ref[...]
    out_ref[...] = jnp.where(mask[None, None, :, :], x, jnp.zeros_like(x))


def kernel(x):
    shape = x.shape
    xs = x.reshape(B, C, F, T)

    row_fc, sums = pl.pallas_call(
        _stats_kernel,
        grid=(G1,),
        in_specs=[pl.BlockSpec((B, C, RB1, T), lambda i: (0, 0, i, 0))],
        out_specs=[
            pl.BlockSpec((B, T), lambda i: (0, 0)),
            pl.BlockSpec((B, 8), lambda i: (0, 0)),
        ],
        out_shape=[
            jax.ShapeDtypeStruct((B, T), jnp.float32),
            jax.ShapeDtypeStruct((B, 8), jnp.float32),
        ],
    )(xs)

    h, band = pl.pallas_call(
        _mask_kernel,
        in_specs=[
            pl.BlockSpec((B, T), lambda: (0, 0)),
            pl.BlockSpec((B, 8), lambda: (0, 0)),
        ],
        out_specs=[
            pl.BlockSpec((1, T), lambda: (0, 0)),
            pl.BlockSpec(memory_space=pltpu.SMEM),
        ],
        out_shape=[
            jax.ShapeDtypeStruct((1, T), jnp.int32),
            jax.ShapeDtypeStruct((2,), jnp.int32),
        ],
    )(row_fc, sums)

    out = pl.pallas_call(
        _apply_kernel,
        grid_spec=pltpu.PrefetchScalarGridSpec(
            num_scalar_prefetch=1,
            grid=(G3,),
            in_specs=[
                pl.BlockSpec((B, C, RB3, T),
                             lambda i, b: (0, 0, jnp.clip(i, b[0], b[1]), 0)),
                pl.BlockSpec((1, T), lambda i, b: (0, 0)),
            ],
            out_specs=pl.BlockSpec((B, C, RB3, T), lambda i, b: (0, 0, i, 0)),
        ),
        out_shape=jax.ShapeDtypeStruct((B, C, F, T), jnp.float32),
    )(band, xs, h)

    return out.reshape(shape)


# speedup vs baseline: 4.1847x; 3.0720x over previous
import jax
import jax.numpy as jnp
from jax import lax
from jax.experimental import pallas as pl
from jax.experimental.pallas import tpu as pltpu

B = 64
F = 599
T = 512
FC_IDX = 219
MID_LO = 199
MID_HI = 239
NOISE_COUNT = (F - (MID_HI - MID_LO + 1)) * T

RF1 = 16
G1 = (F + RF1 - 1) // RF1
RF3 = 16
G3 = (F + RF3 - 1) // RF3


def _stats_kernel(x_ref, rowfc_ref, sums_ref):
    i = pl.program_id(0)
    f = lax.broadcasted_iota(jnp.int32, (RF1, 8, 128), 0) + i * RF1
    a = jnp.abs(x_ref[...])
    zero = jnp.zeros_like(a)
    valid = (f < F)[None]
    midm = ((f >= MID_LO) & (f <= MID_HI))[None]
    fcm = (f == FC_IDX)[None]
    s_all = jnp.sum(jnp.where(valid, a, zero), axis=(1, 2, 3))
    s_mid = jnp.sum(jnp.where(midm, a, zero), axis=(1, 2, 3))
    fc_part = jnp.sum(jnp.where(fcm, a, zero), axis=1)

    @pl.when(i == 0)
    def _():
        rowfc_ref[...] = fc_part
        sums_ref[:, 0:1] = s_all[:, None]
        sums_ref[:, 1:2] = s_mid[:, None]
        sums_ref[:, 2:8] = jnp.zeros((B, 6), jnp.float32)

    @pl.when(i > 0)
    def _():
        rowfc_ref[...] = rowfc_ref[...] + fc_part
        sums_ref[:, 0:1] = sums_ref[:, 0:1] + s_all[:, None]
        sums_ref[:, 1:2] = sums_ref[:, 1:2] + s_mid[:, None]


def _mask_kernel(rowfc_ref, sums_ref, h_ref, band_ref):
    raw = rowfc_ref[...]
    mag = jnp.sum(raw.reshape(B, 4, 2, 128), axis=2)
    t = (lax.broadcasted_iota(jnp.int32, (4, 128), 0) * 128
         + lax.broadcasted_iota(jnp.int32, (4, 128), 1))
    t3 = t[None]
    m = jnp.max(jnp.max(mag, axis=2, keepdims=True), axis=1, keepdims=True)
    midc = jnp.where(mag == m, t3, T)
    mid = jnp.min(jnp.min(midc, axis=2, keepdims=True), axis=1, keepdims=True)
    w20 = (t3 >= jnp.maximum(mid - 20, 0)) & (t3 < mid + 20)
    sigg = jnp.where(w20, mag, jnp.zeros_like(mag))
    sig = jnp.sum(jnp.sum(sigg, axis=2, keepdims=True), axis=1, keepdims=True)
    noise = ((sums_ref[:, 0:1] - sums_ref[:, 1:2])
             * (1.0 / NOISE_COUNT))[:, :, None]
    snr = 10.0 * (jnp.log((sig - noise) ** 2 / noise ** 2) /
                  jnp.log(jnp.float32(10.0)))
    hb = jnp.maximum(jnp.trunc(6.0 * (snr - 48.0) + 27.0).astype(jnp.int32),
                     8)
    cover = (t3 >= mid - 8) & (t3 < jnp.minimum(mid + 8, T))
    h4 = jnp.max(jnp.where(cover, hb, 0), axis=0)
    h_ref[...] = jnp.broadcast_to(h4[:, None, :], (4, 2, 128)).reshape(8, 128)
    hbmax = jnp.max(hb)
    band_lo = jnp.maximum(FC_IDX - hbmax, 0)
    band_hi = jnp.minimum(FC_IDX + hbmax, F)
    band_ref[0] = band_lo // RF3
    band_ref[1] = (band_hi - 1) // RF3


def _apply_kernel(band_ref, x_ref, h_ref, out_ref):
    i = pl.program_id(0)
    f = lax.broadcasted_iota(jnp.int32, (RF3, 8, 128), 0) + i * RF3
    h = h_ref[...][None]
    mask = (f >= FC_IDX - h) & (f < FC_IDX + h)
    x = x_ref[...]
    out_ref[...] = jnp.where(mask[None], x, jnp.zeros_like(x))


def kernel(x):
    shape = x.shape
    xv = (x.reshape(16, 4, 2, F, 4, 128)
          .transpose(0, 1, 3, 4, 2, 5)
          .reshape(B, F, 8, 128))

    row_fc, sums = pl.pallas_call(
        _stats_kernel,
        grid=(G1,),
        in_specs=[pl.BlockSpec((B, RF1, 8, 128), lambda i: (0, i, 0, 0))],
        out_specs=[
            pl.BlockSpec((B, 8, 128), lambda i: (0, 0, 0)),
            pl.BlockSpec((B, 8), lambda i: (0, 0)),
        ],
        out_shape=[
            jax.ShapeDtypeStruct((B, 8, 128), jnp.float32),
            jax.ShapeDtypeStruct((B, 8), jnp.float32),
        ],
    )(xv)

    h, band = pl.pallas_call(
        _mask_kernel,
        in_specs=[
            pl.BlockSpec((B, 8, 128), lambda: (0, 0, 0)),
            pl.BlockSpec((B, 8), lambda: (0, 0)),
        ],
        out_specs=[
            pl.BlockSpec((8, 128), lambda: (0, 0)),
            pl.BlockSpec(memory_space=pltpu.SMEM),
        ],
        out_shape=[
            jax.ShapeDtypeStruct((8, 128), jnp.int32),
            jax.ShapeDtypeStruct((2,), jnp.int32),
        ],
    )(row_fc, sums)

    out = pl.pallas_call(
        _apply_kernel,
        grid_spec=pltpu.PrefetchScalarGridSpec(
            num_scalar_prefetch=1,
            grid=(G3,),
            in_specs=[
                pl.BlockSpec((B, RF3, 8, 128),
                             lambda i, b: (0, jnp.clip(i, b[0], b[1]), 0, 0)),
                pl.BlockSpec((8, 128), lambda i, b: (0, 0)),
            ],
            out_specs=pl.BlockSpec((B, RF3, 8, 128),
                                   lambda i, b: (0, i, 0, 0)),
        ),
        out_shape=jax.ShapeDtypeStruct((B, F, 8, 128), jnp.float32),
    )(band, xv, h)

    return (out.reshape(16, 4, F, 4, 2, 128)
            .transpose(0, 1, 4, 2, 3, 5)
            .reshape(shape))
